# baseline (speedup 1.0000x reference)
import functools

import jax
import jax.numpy as jnp
from jax import lax
from jax.experimental import pallas as pl
from jax.experimental.pallas import tpu as pltpu
from jax.experimental.pallas import tpu_sc as plsc

NC = 2
NS = 16
NW = NC * NS

PAD_D = 128
PACK = 4
ROWS_PER_SLAB = 4
L = 16


def _emb_call(n_tok, s, d):
    mesh = plsc.VectorSubcoreMesh(core_axis_name="c", subcore_axis_name="s")
    rows_per_w = n_tok // NW
    slab_tok = ROWS_PER_SLAB * s
    per_w = rows_per_w * s
    n_slabs = rows_per_w // ROWS_PER_SLAB
    g_sizes = [128, slab_tok - 128]
    n_qv = (slab_tok + L - 1) // L

    @functools.partial(
        pl.kernel,
        mesh=mesh,
        out_type=jax.ShapeDtypeStruct((n_tok, s, d), jnp.float32),
        scratch_types=[
            pltpu.VMEM((per_w,), jnp.int32),
            pltpu.VMEM((n_qv * L,), jnp.int32),
            pltpu.VMEM((n_qv * L,), jnp.int32),
            pltpu.VMEM((2, slab_tok, PAD_D), jnp.float32),
            pltpu.VMEM((ROWS_PER_SLAB, s, d), jnp.float32),
            pltpu.SemaphoreType.DMA,
            pltpu.SemaphoreType.DMA,
            pltpu.SemaphoreType.DMA,
        ],
        compiler_params=pltpu.CompilerParams(use_tc_tiling_on_sc=True),
    )
    def emb(idx_hbm, table_hbm, out_hbm, idx_v, q_v0, q_v1, rows_v, cmp_v,
            g_sem0, g_sem1, s_sem):
        wid = lax.axis_index("s") * NC + lax.axis_index("c")
        base_r = wid * rows_per_w
        pltpu.sync_copy(idx_hbm.at[pl.ds(wid * per_w, per_w)], idx_v)

        def fire(g, b, g_sem, q_v):
            def qbody(i, carry):
                off = lax.min(g * slab_tok + i * L, per_w - L)
                q_v[pl.ds(i * L, L)] = lax.shift_right_logical(
                    idx_v[pl.ds(pl.multiple_of(off, 8), L)], 2
                )
                return carry

            lax.fori_loop(0, n_qv, qbody, 0)
            off = 0
            for sz in g_sizes:
                pltpu.async_copy(
                    table_hbm.at[q_v.at[pl.ds(off, sz)]],
                    rows_v.at[b].at[pl.ds(off, sz)],
                    g_sem,
                )
                off += sz

        def drain(b, g_sem):
            off = 0
            for sz in g_sizes:
                pltpu.make_async_copy(
                    table_hbm.at[pl.ds(0, sz)],
                    rows_v.at[b].at[pl.ds(off, sz)],
                    g_sem,
                ).wait()
                off += sz

        def compact(g, b):
            def group(t0, n_u):
                ids = idx_v[pl.ds(pl.multiple_of(g * slab_tok + t0, 8), L)]
                for u in range(n_u):
                    t = t0 + u
                    r = lax.div(t, s)
                    sp = lax.rem(t, s)
                    lane0 = lax.rem(ids[u], PACK) * d
                    for k in range(d // L):
                        cmp_v[r, sp, pl.ds(k * L, L)] = rows_v[
                            b, t, pl.ds(pl.multiple_of(lane0 + k * L, L), L)
                        ]

            def cbody(j, carry):
                group(j * L, L)
                return carry

            lax.fori_loop(0, slab_tok // L, cbody, 0)
            for t0 in range((slab_tok // L) * L, slab_tok, 8):
                ids8 = idx_v[pl.ds(pl.multiple_of(g * slab_tok + t0, 8), L)]
                for u in range(min(8, slab_tok - t0)):
                    t = t0 + u
                    r = lax.div(t, s)
                    sp = lax.rem(t, s)
                    lane0 = lax.rem(ids8[u], PACK) * d
                    for k in range(d // L):
                        cmp_v[r, sp, pl.ds(k * L, L)] = rows_v[
                            b, t, pl.ds(pl.multiple_of(lane0 + k * L, L), L)
                        ]

        def store(g, s_sem):
            pltpu.async_copy(
                cmp_v,
                out_hbm.at[pl.ds(base_r + g * ROWS_PER_SLAB, ROWS_PER_SLAB)],
                s_sem,
            )

        def wait_store(s_sem):
            pltpu.make_async_copy(
                out_hbm.at[pl.ds(0, ROWS_PER_SLAB)], cmp_v, s_sem
            ).wait()

        fire(0, 0, g_sem0, q_v0)

        def body(g, carry):
            parity = lax.rem(g, 2)

            @pl.when(parity == 0)
            def _even():
                fire(g + 1, 1, g_sem1, q_v1)
                drain(0, g_sem0)

                @pl.when(g >= 1)
                def _():
                    wait_store(s_sem)
                compact(g, 0)
                store(g, s_sem)

            @pl.when(parity == 1)
            def _odd():
                fire(g + 1, 0, g_sem0, q_v0)
                drain(1, g_sem1)
                wait_store(s_sem)
                compact(g, 1)
                store(g, s_sem)

            return carry

        lax.fori_loop(0, n_slabs - 1, body, 0)

        g_last = n_slabs - 1
        b_last = g_last % 2
        drain(b_last, g_sem0 if b_last == 0 else g_sem1)
        if n_slabs > 1:
            wait_store(s_sem)
        compact(g_last, b_last)
        store(g_last, s_sem)
        wait_store(s_sem)

    return emb


def kernel(token_ids, weight):
    n_tok, s = token_ids.shape
    n, d = weight.shape
    tok1 = token_ids.reshape(-1)
    w_lines = weight.reshape(n * d // PAD_D, PAD_D)
    return _emb_call(n_tok, s, d)(tok1, w_lines)


# speedup vs baseline: 1.1003x; 1.1003x over previous
import functools

import jax
import jax.numpy as jnp
from jax import lax
from jax.experimental import pallas as pl
from jax.experimental.pallas import tpu as pltpu
from jax.experimental.pallas import tpu_sc as plsc

NC = 2
NS = 16
NW = NC * NS

RPS = 16


def _emb_call(n_tok, s, d):
    mesh = plsc.VectorSubcoreMesh(core_axis_name="c", subcore_axis_name="s")
    rows_per_w = n_tok // NW
    n_slabs = rows_per_w // RPS

    @functools.partial(
        pl.kernel,
        mesh=mesh,
        out_type=jax.ShapeDtypeStruct((n_tok, s, d), jnp.float32),
        scratch_types=[
            pltpu.VMEM((rows_per_w, s), jnp.int32),
            pltpu.VMEM((2, RPS, s, d), jnp.float32),
            pltpu.SemaphoreType.DMA,
            pltpu.SemaphoreType.DMA,
            pltpu.SemaphoreType.DMA,
            pltpu.SemaphoreType.DMA,
        ],
        compiler_params=pltpu.CompilerParams(use_tc_tiling_on_sc=False),
    )
    def emb(idx_hbm, table_hbm, out_hbm, idx_v, rows_v, g_sem0, g_sem1,
            s_sem0, s_sem1):
        wid = lax.axis_index("s") * NC + lax.axis_index("c")
        base = wid * rows_per_w
        pltpu.sync_copy(idx_hbm.at[pl.ds(base, rows_per_w)], idx_v)

        def fire(g, b, g_sem):
            for j in range(RPS):
                pltpu.async_copy(
                    table_hbm.at[idx_v.at[g * RPS + j]],
                    rows_v.at[b].at[j],
                    g_sem,
                )

        def drain(b, g_sem):
            pltpu.make_async_copy(
                out_hbm.at[pl.ds(0, RPS)], rows_v.at[b], g_sem
            ).wait()

        def store(g, b, s_sem):
            pltpu.async_copy(
                rows_v.at[b], out_hbm.at[pl.ds(base + g * RPS, RPS)], s_sem
            )

        def wait_store(b, s_sem):
            pltpu.make_async_copy(
                out_hbm.at[pl.ds(0, RPS)], rows_v.at[b], s_sem
            ).wait()

        fire(0, 0, g_sem0)

        def body(g, carry):
            b = lax.rem(g, 2)

            @pl.when(b == 0)
            def _even():
                @pl.when(g >= 1)
                def _():
                    wait_store(1, s_sem1)
                fire(g + 1, 1, g_sem1)
                drain(0, g_sem0)
                store(g, 0, s_sem0)

            @pl.when(b == 1)
            def _odd():
                wait_store(0, s_sem0)
                fire(g + 1, 0, g_sem0)
                drain(1, g_sem1)
                store(g, 1, s_sem1)

            return carry

        lax.fori_loop(0, n_slabs - 1, body, 0)

        g_last = n_slabs - 1
        b_last = g_last % 2
        g_sem_last = g_sem0 if b_last == 0 else g_sem1
        s_sem_last = s_sem0 if b_last == 0 else s_sem1
        drain(b_last, g_sem_last)
        if n_slabs > 1:
            b_prev = 1 - b_last
            wait_store(b_prev, s_sem0 if b_prev == 0 else s_sem1)
        store(g_last, b_last, s_sem_last)
        wait_store(b_last, s_sem_last)

    return emb


def kernel(token_ids, weight):
    n_tok, s = token_ids.shape
    n, d = weight.shape
    return _emb_call(n_tok, s, d)(token_ids, weight)
